# Initial kernel scaffold; baseline (speedup 1.0000x reference)
import jax
import jax.numpy as jnp
from jax.experimental import pallas as pl


def kernel(p0, p1, p2, gt_bboxes, gt_labels):
    raise NotImplementedError("write your pallas kernel here")


# speedup vs baseline: 15.9237x; 15.9237x over previous
import functools

import jax
import jax.numpy as jnp
import numpy as np
from jax.experimental import pallas as pl
from jax.experimental.pallas import tpu as pltpu

N0, N1, N2 = 64 * 64, 32 * 32, 16 * 16
N = N0 + N1 + N2
G = 20
C = 80
BINS = 16
TOPK = 10
EPS = 1e-9
W_CLS, W_IOU, W_DFL = 0.5, 7.5, 1.5


def _make_geo():
    rows = []
    for (h, w, s) in ((64, 64, 8.0), (32, 32, 16.0), (16, 16, 32.0)):
        yy, xx = np.meshgrid(np.arange(h, dtype=np.float32),
                             np.arange(w, dtype=np.float32), indexing="ij")
        cx = ((xx + 0.5) * s).reshape(-1)
        cy = ((yy + 0.5) * s).reshape(-1)
        st = np.full(h * w, s, np.float32)
        rows.append(np.stack([cx, cy, st], 0))
    geo = np.concatenate(rows, axis=1)
    return np.concatenate([geo, np.zeros((5, N), np.float32)], axis=0)


_GEO = _make_geo()

_ATAN_C = (0.9999999937538802, -0.33333137974716015, 0.19993694319366187,
           -0.14211106054436182, 0.10667486902233639, -0.07556900202632058,
           0.043278241746605356, -0.01641319040050145, 0.0029327619377836774)


def _atan(x):
    ax = jnp.abs(x)
    inv = ax > 1.0
    t = jnp.where(inv, 1.0 / jnp.maximum(ax, 1e-30), ax)
    u = t * t
    p = jnp.float32(_ATAN_C[-1])
    for c in _ATAN_C[-2::-1]:
        p = p * u + c
    r = t * p
    r = jnp.where(inv, jnp.float32(np.pi / 2) - r, r)
    return jnp.where(x < 0, -r, r)


def _loss_kernel(geo_ref, p0_ref, p1_ref, p2_ref, gtb_ref, gtl_ref, out_ref):
    cx = geo_ref[0:1, :]
    cy = geo_ref[1:2, :]
    stride = geo_ref[2:3, :]

    x = jnp.concatenate([p0_ref[0, 64:64 + C, :], p1_ref[0, 64:64 + C, :],
                         p2_ref[0, 64:64 + C, :]], axis=1)
    d = jnp.concatenate([p0_ref[0, 0:64, :], p1_ref[0, 0:64, :],
                         p2_ref[0, 0:64, :]], axis=1)

    proj = jax.lax.broadcasted_iota(jnp.int32, (BINS, 1), 0).astype(jnp.float32)
    dists = []
    logz = []
    for s in range(4):
        blk = d[BINS * s:BINS * (s + 1), :]
        m = jnp.max(blk, axis=0, keepdims=True)
        e = jnp.exp(blk - m)
        se = jnp.sum(e, axis=0, keepdims=True)
        prob = e / se
        dists.append(jnp.sum(prob * proj, axis=0, keepdims=True) * stride)
        logz.append(m + jnp.log(se))
    bx1 = cx - dists[0]
    by1 = cy - dists[1]
    bx2 = cx + dists[2]
    by2 = cy + dists[3]

    gtb = gtb_ref[0]
    gx1 = gtb[:, 0:1]
    gy1 = gtb[:, 1:2]
    gx2 = gtb[:, 2:3]
    gy2 = gtb[:, 3:4]
    lbl = jnp.clip(gtl_ref[0], 0, C - 1)

    lbl_oh = (lbl == jax.lax.broadcasted_iota(jnp.int32, (G, C), 1)
              ).astype(jnp.float32)
    cls_g = jnp.dot(lbl_oh, x, preferred_element_type=jnp.float32)
    cls_s = jax.nn.sigmoid(cls_g)

    ix1 = jnp.maximum(gx1, bx1)
    iy1 = jnp.maximum(gy1, by1)
    ix2 = jnp.minimum(gx2, bx2)
    iy2 = jnp.minimum(gy2, by2)
    inter = jnp.clip(ix2 - ix1, 0) * jnp.clip(iy2 - iy1, 0)
    ag = (gx2 - gx1) * (gy2 - gy1)
    ap = (bx2 - bx1) * (by2 - by1)
    union = ag + ap - inter
    iou = inter / (union + EPS)

    iou2 = iou * iou
    iou6 = iou2 * iou2 * iou2
    align = jnp.sqrt(cls_s) * iou6

    pcx = (bx1 + bx2) * 0.5
    pcy = (by1 + by2) * 0.5
    in_gt = (pcx >= gx1) & (pcx < gx2) & (pcy >= gy1) & (pcy < gy2)
    valid = ((gx2 - gx1) > 0) & ((gy2 - gy1) > 0)
    mask = in_gt & valid
    metric = jnp.where(mask, align, 0.0)

    lane = jax.lax.broadcasted_iota(jnp.int32, (G, N), 1)
    picked = jnp.zeros((G, N), jnp.bool_)
    mwork = metric
    for _ in range(TOPK):
        mv = jnp.max(mwork, axis=1, keepdims=True)
        idx = jnp.min(jnp.where(mwork == mv, lane, N), axis=1,
                      keepdims=True)
        sel = lane == idx
        picked = picked | (sel & (mv > EPS))
        mwork = jnp.where(sel, -1.0, mwork)

    mp = (picked & mask).astype(jnp.float32)
    fg_count = jnp.sum(mp, axis=0, keepdims=True)

    g_iota = jax.lax.broadcasted_iota(jnp.int32, (G, N), 0)
    x_iou = jnp.where(mp > 0, iou, -1.0)
    mxv = jnp.max(x_iou, axis=0, keepdims=True)
    gsel = jnp.min(jnp.where(x_iou == mxv, g_iota, G), axis=0,
                   keepdims=True)
    is_max = (g_iota == gsel).astype(jnp.float32)
    mp = jnp.where(fg_count > 1, is_max * mp, mp)

    fg = jnp.sum(mp, axis=0, keepdims=True) > 0
    fg_f = fg.astype(jnp.float32)
    gmin = jnp.min(jnp.where(mp > 0, g_iota, G), axis=0, keepdims=True)
    gmatch = jnp.where(fg, gmin, 0)
    moh = (g_iota == gmatch).astype(jnp.float32)

    tx1 = jnp.sum(moh * gx1, axis=0, keepdims=True)
    ty1 = jnp.sum(moh * gy1, axis=0, keepdims=True)
    tx2 = jnp.sum(moh * gx2, axis=0, keepdims=True)
    ty2 = jnp.sum(moh * gy2, axis=0, keepdims=True)
    xsel = jnp.sum(moh * cls_g, axis=0, keepdims=True)

    align_m = align * mp
    pos_align = jnp.max(align_m, axis=1, keepdims=True)
    pos_iou = jnp.max(iou * mp, axis=1, keepdims=True)
    norm = jnp.max(align_m * pos_iou / (pos_align + EPS), axis=0,
                   keepdims=True)
    w = norm * fg_f

    base = jnp.sum(jnp.maximum(x, 0.0) + jnp.log1p(jnp.exp(-jnp.abs(x))))
    cls_sum = base - jnp.sum(w * xsel)

    cix1 = jnp.maximum(bx1, tx1)
    ciy1 = jnp.maximum(by1, ty1)
    cix2 = jnp.minimum(bx2, tx2)
    ciy2 = jnp.minimum(by2, ty2)
    cinter = jnp.clip(cix2 - cix1, 0) * jnp.clip(ciy2 - ciy1, 0)
    at = (tx2 - tx1) * (ty2 - ty1)
    cunion = ap + at - cinter
    ciou = cinter / (cunion + EPS)
    ex1 = jnp.minimum(bx1, tx1)
    ey1 = jnp.minimum(by1, ty1)
    ex2 = jnp.maximum(bx2, tx2)
    ey2 = jnp.maximum(by2, ty2)
    c2 = (ex2 - ex1) ** 2 + (ey2 - ey1) ** 2 + EPS
    rho2 = ((bx1 + bx2 - tx1 - tx2) ** 2 + (by1 + by2 - ty1 - ty2) ** 2) / 4.0
    wp = bx2 - bx1
    hp = by2 - by1 + EPS
    wt = tx2 - tx1
    ht = ty2 - ty1 + EPS
    v = (4.0 / (np.pi ** 2)) * (_atan(wt / ht) - _atan(wp / hp)) ** 2
    a = v / (v - ciou + 1.0 + EPS)
    lci = 1.0 - (ciou - rho2 / c2 - a * v)
    iou_sum = jnp.sum(lci * fg_f)

    dfl_sum = jnp.float32(0.0)
    tdists = (jnp.clip(cx - tx1, 0), jnp.clip(cy - ty1, 0),
              jnp.clip(tx2 - cx, 0), jnp.clip(ty2 - cy, 0))
    b_iota = jax.lax.broadcasted_iota(jnp.int32, (BINS, N), 0)
    for s in range(4):
        tb = jnp.clip(tdists[s], 0.0, BINS - 1 - 1e-6)
        lo = jnp.floor(tb)
        al = tb - lo
        lo_i = lo.astype(jnp.int32)
        up_i = jnp.clip(lo_i + 1, 0, BINS - 1)
        blk = d[BINS * s:BINS * (s + 1), :]
        val_lo = jnp.sum(jnp.where(b_iota == lo_i, blk, 0.0), axis=0,
                         keepdims=True)
        val_up = jnp.sum(jnp.where(b_iota == up_i, blk, 0.0), axis=0,
                         keepdims=True)
        nll = (1.0 - al) * (logz[s] - val_lo) + al * (logz[s] - val_up)
        dfl_sum = dfl_sum + jnp.sum(fg_f * nll)

    np_sum = jnp.sum(fg_f)

    out_ref[0, 0, 0] = iou_sum
    out_ref[0, 0, 1] = cls_sum
    out_ref[0, 0, 2] = dfl_sum
    out_ref[0, 0, 3] = np_sum


@functools.partial(jax.jit, static_argnames=())
def kernel(p0, p1, p2, gt_bboxes, gt_labels):
    B = p0.shape[0]
    p0r = p0.reshape(B, 144, N0)
    p1r = p1.reshape(B, 144, N1)
    p2r = p2.reshape(B, 144, N2)
    gtl = gt_labels.astype(jnp.int32).reshape(B, G, 1)
    geo = jnp.asarray(_GEO)

    parts = pl.pallas_call(
        _loss_kernel,
        grid=(B,),
        in_specs=[
            pl.BlockSpec((8, N), lambda b: (0, 0)),
            pl.BlockSpec((1, 144, N0), lambda b: (b, 0, 0)),
            pl.BlockSpec((1, 144, N1), lambda b: (b, 0, 0)),
            pl.BlockSpec((1, 144, N2), lambda b: (b, 0, 0)),
            pl.BlockSpec((1, G, 4), lambda b: (b, 0, 0)),
            pl.BlockSpec((1, G, 1), lambda b: (b, 0, 0)),
        ],
        out_specs=pl.BlockSpec((1, 1, 4), lambda b: (b, 0, 0),
                               memory_space=pltpu.SMEM),
        out_shape=jax.ShapeDtypeStruct((B, 1, 4), jnp.float32),
    )(geo, p0r, p1r, p2r, gt_bboxes, gtl)

    sums = parts.sum(axis=(0, 1))
    denom = jnp.maximum(1.0, sums[3])
    return jnp.stack([W_IOU * sums[0], W_CLS * sums[1],
                      W_DFL * sums[2]]) / denom
